# baseline (speedup 1.0000x reference)
import functools

import jax
import jax.numpy as jnp
import numpy as np
from jax import lax
from jax.experimental import pallas as pl
from jax.experimental.pallas import tpu as pltpu
from jax.experimental.pallas import tpu_sc as plsc

N = 16384 * 512
R = 16384
D = 512
RW = R // 32
SR = 24
NW = 32
CW = N // NW
C = 8192
T = CW // C
NG = C // 64
NSUB = NW * T
NCV = NSUB // 16
NB = 4

CA = 65536
CAW = CA // 4
CWW = CW // 4
CMW = C // 4
TA = CW // CA
QA = CA // C
BR = 2048


def _worker_id():
    return lax.axis_index("s") * 2 + lax.axis_index("c")


def _bits_of(w):
    b0 = w & 1
    b1 = lax.shift_right_logical(w, 1) & 1
    b2 = lax.shift_right_logical(w, 2) & 1
    b3 = lax.shift_right_logical(w, 3) & 1
    return b0, b1, b2, b3


def _pack_count_kernel(mask_ref, p_ref, packed_ref, counts_ref):
    mf = mask_ref[...].astype(jnp.float32)
    w = jax.lax.dot_general(
        mf, p_ref[...], (((1,), (0,)), ((), ())),
        preferred_element_type=jnp.float32,
    )
    packed_ref[...] = w.astype(jnp.int32)
    c = jnp.sum(mf.reshape(BR // 16, 16, D), axis=(1, 2))
    counts_ref[...] = c.astype(jnp.int32).reshape(BR // 256, 16)


def _scatter_kernel(mask_hbm, input_hbm, source_hbm, counts_hbm, out_hbm,
                    mbuf0, mbuf1, mbuf2, mbuf3,
                    sbuf0, sbuf1, sbuf2, sbuf3,
                    obuf0, obuf1, obuf2, obuf3,
                    cbuf, lut,
                    sem0, sem1, sem2, sem3):
    wid = _worker_id()
    lane = lax.iota(jnp.int32, 16)
    lane4 = lane * 4
    mbufs = (mbuf0, mbuf1, mbuf2, mbuf3)
    sbufs = (sbuf0, sbuf1, sbuf2, sbuf3)
    obufs = (obuf0, obuf1, obuf2, obuf3)
    sems = (sem0, sem1, sem2, sem3)

    pltpu.sync_copy(counts_hbm, cbuf)

    t1 = lane & 1
    t2 = t1 + (lax.shift_right_logical(lane, 1) & 1)
    t3 = t2 + (lax.shift_right_logical(lane, 2) & 1)
    t4 = t3 + (lax.shift_right_logical(lane, 3) & 1)
    lut[0, :] = t4
    lut[1, :] = t2
    lut[2, :] = t3

    lim = wid * T

    def accw(j, s):
        vec = cbuf[j, :]
        gidx = j * 16 + lane
        return s + jnp.sum(jnp.where(gidx < lim, vec, 0))

    base0 = lax.fori_loop(0, NCV, accw, jnp.int32(0))

    def cntof(t):
        j = jnp.minimum(wid * T + t, NSUB - 1)
        vec = cbuf[j // 16, :]
        return jnp.sum(jnp.where(lane == j % 16, vec, 0))

    def srow_of(b):
        return jnp.minimum(lax.shift_right_logical(b, 12) * 8, R - SR)

    def issue_in(t, base_t, b):
        rr = wid * RW + t * 16
        pltpu.async_copy(mask_hbm.at[pl.ds(rr, 16), :], mbufs[b],
                         sems[b])
        pltpu.async_copy(input_hbm.at[pl.ds(rr, 16), :], obufs[b],
                         sems[b])
        pltpu.async_copy(source_hbm.at[pl.ds(srow_of(base_t), SR), :],
                         sbufs[b], sems[b])

    def wait_in(t, b):
        rr = wid * RW + t * 16
        pltpu.make_async_copy(mask_hbm.at[pl.ds(0, 16), :], mbufs[b],
                              sems[b]).wait()
        pltpu.make_async_copy(input_hbm.at[pl.ds(rr, 16), :], obufs[b],
                              sems[b]).wait()
        pltpu.make_async_copy(source_hbm.at[pl.ds(0, SR), :], sbufs[b],
                              sems[b]).wait()

    def wait_out(t, b):
        rr = wid * RW + t * 16
        pltpu.make_async_copy(obufs[b], out_hbm.at[pl.ds(rr, 16), :],
                              sems[b]).wait()

    base1 = base0 + cntof(0)
    base2 = base1 + cntof(1)
    issue_in(0, base0, 0)
    issue_in(1, base1, 1)

    def step(t, b, carry):
        base_cur, base_next, base_next2 = carry
        mbuf, sbuf, obuf = mbufs[b], sbufs[b], obufs[b]
        wait_in(t, b)

        off = base_cur - srow_of(base_cur) * D

        zero16 = jnp.zeros((16,), jnp.int32)

        def inner(g, carry):
            w = mbuf[lax.shift_right_logical(g, 3),
                     pl.ds((g & 7) * 16, 16)]
            b0 = w & 1
            s4 = plsc.load_gather(lut, [zero16, w])
            e2 = plsc.load_gather(lut, [zero16 + 1, w])
            e3 = plsc.load_gather(lut, [zero16 + 2, w])
            inc = plsc.cumsum(s4)
            r0 = jnp.full((16,), carry, jnp.int32) + (inc - s4)
            r1 = r0 + b0
            r2 = r0 + e2
            r3 = r0 + e3
            orow = jnp.full((16,), lax.shift_right_logical(g, 3), jnp.int32)
            ocol = lane4 + (g & 7) * 64

            def gat(rk):
                return plsc.load_gather(
                    sbuf, [lax.shift_right_logical(rk, 9), rk & (D - 1)])

            v0 = gat(r0)
            v1 = gat(r1)
            v2 = gat(r2)
            v3 = gat(r3)
            plsc.store_scatter(obuf, [orow, ocol], v0, mask=b0 != 0)
            plsc.store_scatter(obuf, [orow, ocol + 1], v1,
                               mask=(w & 2) != 0)
            plsc.store_scatter(obuf, [orow, ocol + 2], v2,
                               mask=(w & 4) != 0)
            plsc.store_scatter(obuf, [orow, ocol + 3], v3,
                               mask=(w & 8) != 0)
            return carry + inc[15]

        plsc.parallel_loop(0, NG, unroll=8, carry=off)(inner)

        rr = wid * RW + t * 16
        pltpu.async_copy(obuf, out_hbm.at[pl.ds(rr, 16), :], sems[b])

        base_next3 = base_next2 + cntof(t + 2)

        @pl.when(t + 2 < T)
        def _():
            @pl.when(t >= 2)
            def _():
                wait_out(t - 2, (b + 2) % NB)

            issue_in(t + 2, base_next2, (b + 2) % NB)

        return base_next, base_next2, base_next3

    def quad(tq, carry):
        for b in range(NB):
            carry = step(tq * NB + b, b, carry)
        return carry

    lax.fori_loop(0, T // NB, quad, (base0, base1, base2))
    wait_out(T - 4, (T - 4) % NB)
    wait_out(T - 3, (T - 3) % NB)
    wait_out(T - 2, (T - 2) % NB)
    wait_out(T - 1, (T - 1) % NB)


def kernel(input, mask, source):
    mesh = plsc.VectorSubcoreMesh(core_axis_name="c", subcore_axis_name="s")
    params = pltpu.CompilerParams(needs_layout_passes=False)

    pmat = np.zeros((D, D // 4), np.float32)
    for j in range(D):
        pmat[j, j // 4] = float(1 << (j % 4))
    pmat = jnp.asarray(pmat)
    packed, counts = pl.pallas_call(
        _pack_count_kernel,
        grid=(R // BR,),
        in_specs=[
            pl.BlockSpec((BR, D), lambda i: (i, 0)),
            pl.BlockSpec((D, D // 4), lambda i: (0, 0)),
        ],
        out_specs=[
            pl.BlockSpec((BR, D // 4), lambda i: (i, 0)),
            pl.BlockSpec((BR // 256, 16), lambda i: (i, 0)),
        ],
        out_shape=[
            jax.ShapeDtypeStruct((R, D // 4), jnp.int32),
            jax.ShapeDtypeStruct((NSUB // 16, 16), jnp.int32),
        ],
    )(mask, pmat)

    scatter_call = functools.partial(
        pl.kernel,
        mesh=mesh,
        compiler_params=params,
        out_type=jax.ShapeDtypeStruct((R, D), jnp.float32),
        scratch_types=(
            [pltpu.VMEM((16, D // 4), jnp.int32)] * 4
            + [pltpu.VMEM((SR, D), jnp.float32)] * 4
            + [pltpu.VMEM((16, D), jnp.float32)] * 4
            + [pltpu.VMEM((NSUB // 16, 16), jnp.int32)]
            + [pltpu.VMEM((4, 16), jnp.int32)]
            + [pltpu.SemaphoreType.DMA] * 4
        ),
    )(_scatter_kernel)
    return scatter_call(packed, input, source, counts)


# speedup vs baseline: 1.0246x; 1.0246x over previous
import functools

import jax
import jax.numpy as jnp
import numpy as np
from jax import lax
from jax.experimental import pallas as pl
from jax.experimental.pallas import tpu as pltpu
from jax.experimental.pallas import tpu_sc as plsc

N = 16384 * 512
R = 16384
D = 512
RW = R // 32
SR = 24
NW = 32
CW = N // NW
C = 8192
T = CW // C
NG = C // 64
NSUB = NW * T
NCV = NSUB // 16
NB = 4

CA = 65536
CAW = CA // 4
CWW = CW // 4
CMW = C // 4
TA = CW // CA
QA = CA // C
BR = 2048


def _worker_id():
    return lax.axis_index("s") * 2 + lax.axis_index("c")


def _bits_of(w):
    b0 = w & 1
    b1 = lax.shift_right_logical(w, 1) & 1
    b2 = lax.shift_right_logical(w, 2) & 1
    b3 = lax.shift_right_logical(w, 3) & 1
    return b0, b1, b2, b3


def _pack_count_kernel(mask_ref, p_ref, packed_ref, counts_ref):
    mf = mask_ref[...].astype(jnp.float32)
    w = jax.lax.dot_general(
        mf, p_ref[...], (((1,), (0,)), ((), ())),
        preferred_element_type=jnp.float32,
    )
    packed_ref[...] = w.astype(jnp.int32)
    c = jnp.sum(mf.reshape(BR // 16, 16, D), axis=(1, 2))
    counts_ref[...] = c.astype(jnp.int32).reshape(BR // 256, 16)


def _scatter_kernel(mask_hbm, input_hbm, source_hbm, counts_hbm, out_hbm,
                    mbuf0, mbuf1, mbuf2, mbuf3,
                    sbuf0, sbuf1, sbuf2, sbuf3,
                    obuf0, obuf1, obuf2, obuf3,
                    cbuf, lut,
                    sem_m0, sem_m1, sem_m2, sem_m3,
                    sem_s0, sem_s1, sem_s2, sem_s3,
                    sem_i0, sem_i1, sem_i2, sem_i3,
                    sem_o0, sem_o1, sem_o2, sem_o3):
    wid = _worker_id()
    lane = lax.iota(jnp.int32, 16)
    lane4 = lane * 4
    mbufs = (mbuf0, mbuf1, mbuf2, mbuf3)
    sbufs = (sbuf0, sbuf1, sbuf2, sbuf3)
    obufs = (obuf0, obuf1, obuf2, obuf3)
    sems_m = (sem_m0, sem_m1, sem_m2, sem_m3)
    sems_s = (sem_s0, sem_s1, sem_s2, sem_s3)
    sems_i = (sem_i0, sem_i1, sem_i2, sem_i3)
    sems_o = (sem_o0, sem_o1, sem_o2, sem_o3)

    pltpu.sync_copy(counts_hbm, cbuf)

    t1 = lane & 1
    t2 = t1 + (lax.shift_right_logical(lane, 1) & 1)
    t3 = t2 + (lax.shift_right_logical(lane, 2) & 1)
    t4 = t3 + (lax.shift_right_logical(lane, 3) & 1)
    lut[0, :] = t4
    lut[1, :] = t2
    lut[2, :] = t3

    lim = wid * T

    def accw(j, s):
        vec = cbuf[j, :]
        gidx = j * 16 + lane
        return s + jnp.sum(jnp.where(gidx < lim, vec, 0))

    base0 = lax.fori_loop(0, NCV, accw, jnp.int32(0))

    def cntof(t):
        j = jnp.minimum(wid * T + t, NSUB - 1)
        vec = cbuf[j // 16, :]
        return jnp.sum(jnp.where(lane == j % 16, vec, 0))

    def srow_of(b):
        return jnp.minimum(lax.shift_right_logical(b, 12) * 8, R - SR)

    def issue_in(t, base_t, b):
        rr = wid * RW + t * 16
        pltpu.async_copy(mask_hbm.at[pl.ds(rr, 16), :], mbufs[b],
                         sems_m[b])
        pltpu.async_copy(input_hbm.at[pl.ds(rr, 16), :], obufs[b],
                         sems_i[b])
        pltpu.async_copy(source_hbm.at[pl.ds(srow_of(base_t), SR), :],
                         sbufs[b], sems_s[b])

    def wait_in(t, b):
        rr = wid * RW + t * 16
        pltpu.make_async_copy(mask_hbm.at[pl.ds(0, 16), :], mbufs[b],
                              sems_m[b]).wait()
        pltpu.make_async_copy(input_hbm.at[pl.ds(rr, 16), :], obufs[b],
                              sems_i[b]).wait()
        pltpu.make_async_copy(source_hbm.at[pl.ds(0, SR), :], sbufs[b],
                              sems_s[b]).wait()

    def wait_out(t, b):
        rr = wid * RW + t * 16
        pltpu.make_async_copy(obufs[b], out_hbm.at[pl.ds(rr, 16), :],
                              sems_o[b]).wait()

    base1 = base0 + cntof(0)
    base2 = base1 + cntof(1)
    issue_in(0, base0, 0)
    issue_in(1, base1, 1)

    def step(t, b, carry):
        base_cur, base_next, base_next2 = carry
        mbuf, sbuf, obuf = mbufs[b], sbufs[b], obufs[b]
        wait_in(t, b)

        off = base_cur - srow_of(base_cur) * D

        zero16 = jnp.zeros((16,), jnp.int32)

        def inner(g, carry):
            w = mbuf[lax.shift_right_logical(g, 3),
                     pl.ds((g & 7) * 16, 16)]
            b0 = w & 1
            s4 = plsc.load_gather(lut, [zero16, w])
            e2 = plsc.load_gather(lut, [zero16 + 1, w])
            e3 = plsc.load_gather(lut, [zero16 + 2, w])
            inc = plsc.cumsum(s4)
            r0 = jnp.full((16,), carry, jnp.int32) + (inc - s4)
            r1 = r0 + b0
            r2 = r0 + e2
            r3 = r0 + e3
            orow = jnp.full((16,), lax.shift_right_logical(g, 3), jnp.int32)
            ocol = lane4 + (g & 7) * 64

            def gat(rk):
                return plsc.load_gather(
                    sbuf, [lax.shift_right_logical(rk, 9), rk & (D - 1)])

            v0 = gat(r0)
            v1 = gat(r1)
            v2 = gat(r2)
            v3 = gat(r3)
            plsc.store_scatter(obuf, [orow, ocol], v0, mask=b0 != 0)
            plsc.store_scatter(obuf, [orow, ocol + 1], v1,
                               mask=(w & 2) != 0)
            plsc.store_scatter(obuf, [orow, ocol + 2], v2,
                               mask=(w & 4) != 0)
            plsc.store_scatter(obuf, [orow, ocol + 3], v3,
                               mask=(w & 8) != 0)
            return carry + inc[15]

        plsc.parallel_loop(0, NG, unroll=8, carry=off)(inner)

        rr = wid * RW + t * 16
        pltpu.async_copy(obuf, out_hbm.at[pl.ds(rr, 16), :], sems_o[b])

        base_next3 = base_next2 + cntof(t + 2)

        @pl.when(t + 2 < T)
        def _():
            @pl.when(t >= 2)
            def _():
                wait_out(t - 2, (b + 2) % NB)

            issue_in(t + 2, base_next2, (b + 2) % NB)

        return base_next, base_next2, base_next3

    def quad(tq, carry):
        for b in range(NB):
            carry = step(tq * NB + b, b, carry)
        return carry

    lax.fori_loop(0, T // NB, quad, (base0, base1, base2))
    wait_out(T - 4, (T - 4) % NB)
    wait_out(T - 3, (T - 3) % NB)
    wait_out(T - 2, (T - 2) % NB)
    wait_out(T - 1, (T - 1) % NB)


def kernel(input, mask, source):
    mesh = plsc.VectorSubcoreMesh(core_axis_name="c", subcore_axis_name="s")
    params = pltpu.CompilerParams(needs_layout_passes=False)

    pmat = np.zeros((D, D // 4), np.float32)
    for j in range(D):
        pmat[j, j // 4] = float(1 << (j % 4))
    pmat = jnp.asarray(pmat)
    packed, counts = pl.pallas_call(
        _pack_count_kernel,
        grid=(R // BR,),
        in_specs=[
            pl.BlockSpec((BR, D), lambda i: (i, 0)),
            pl.BlockSpec((D, D // 4), lambda i: (0, 0)),
        ],
        out_specs=[
            pl.BlockSpec((BR, D // 4), lambda i: (i, 0)),
            pl.BlockSpec((BR // 256, 16), lambda i: (i, 0)),
        ],
        out_shape=[
            jax.ShapeDtypeStruct((R, D // 4), jnp.int32),
            jax.ShapeDtypeStruct((NSUB // 16, 16), jnp.int32),
        ],
    )(mask, pmat)

    scatter_call = functools.partial(
        pl.kernel,
        mesh=mesh,
        compiler_params=params,
        out_type=jax.ShapeDtypeStruct((R, D), jnp.float32),
        scratch_types=(
            [pltpu.VMEM((16, D // 4), jnp.int32)] * 4
            + [pltpu.VMEM((SR, D), jnp.float32)] * 4
            + [pltpu.VMEM((16, D), jnp.float32)] * 4
            + [pltpu.VMEM((NSUB // 16, 16), jnp.int32)]
            + [pltpu.VMEM((4, 16), jnp.int32)]
            + [pltpu.SemaphoreType.DMA] * 16
        ),
    )(_scatter_kernel)
    return scatter_call(packed, input, source, counts)
